# baseline (speedup 1.0000x reference)
import jax
import jax.numpy as jnp
from jax.experimental import pallas as pl
from jax.experimental.pallas import tpu as pltpu


def _dma_kernel(x_ref, w_ref, b_ref, o_ref, y_ref, sem):
    n = x_ref.shape[0]
    h = n // 2

    def half(lo):
        y = jax.lax.dot_general(
            x_ref[pl.ds(lo, h), :], w_ref[...], (((1,), (1,)), ((), ())),
            preferred_element_type=jnp.float32)
        y_ref[pl.ds(lo, h), :] = jnp.maximum(y + b_ref[...], 0.0)

    def issue(i, lo):
        return pltpu.make_async_copy(
            y_ref.at[pl.ds(lo, h), :], o_ref.at[i, pl.ds(lo, h), :], sem)

    half(0)

    def fire0(i, _):
        issue(i, 0).start()
        return 0

    jax.lax.fori_loop(0, n, fire0, 0)
    half(h)

    def fire1(i, _):
        issue(i, h).start()
        return 0

    jax.lax.fori_loop(0, n, fire1, 0)

    def drain(i, _):
        issue(i, 0).wait()
        issue(i, h).wait()
        return 0

    jax.lax.fori_loop(0, n, drain, 0)


def kernel(X, sample_domain, W, b):
    n, d = X.shape
    out = pl.pallas_call(
        _dma_kernel,
        in_specs=[
            pl.BlockSpec(memory_space=pltpu.VMEM),
            pl.BlockSpec(memory_space=pltpu.VMEM),
            pl.BlockSpec(memory_space=pltpu.VMEM),
        ],
        out_specs=pl.BlockSpec(memory_space=pl.ANY),
        out_shape=jax.ShapeDtypeStruct((n, n, d), jnp.float32),
        scratch_shapes=[
            pltpu.VMEM((n, d), jnp.float32),
            pltpu.SemaphoreType.DMA,
        ],
    )(X, W, b.reshape(1, d))
    return out


# speedup vs baseline: 1.0176x; 1.0176x over previous
import jax
import jax.numpy as jnp
from jax.experimental import pallas as pl
from jax.experimental.pallas import tpu as pltpu

_CHUNK = 8


_REP = 1


def _dma_kernel(x_ref, w_ref, b_ref, o_ref, y_ref, sem):
    y = jax.lax.dot_general(
        x_ref[...], w_ref[...], (((1,), (1,)), ((), ())),
        preferred_element_type=jnp.float32)
    yr = jnp.maximum(y + b_ref[...], 0.0)
    y_ref[...] = jnp.broadcast_to(yr[None], y_ref.shape)
    n = x_ref.shape[0]

    def issue(i):
        return pltpu.make_async_copy(
            y_ref, o_ref.at[pl.ds(i * _REP, _REP)], sem)

    ncopies = n // _REP
    for j in range(_CHUNK):
        issue(j).start()

    def body(i, _):
        issue(i + _CHUNK).start()
        issue(i).wait()
        return 0

    jax.lax.fori_loop(0, ncopies - _CHUNK, body, 0)
    for j in range(_CHUNK):
        issue(ncopies - _CHUNK + j).wait()


def kernel(X, sample_domain, W, b):
    n, d = X.shape
    out = pl.pallas_call(
        _dma_kernel,
        in_specs=[
            pl.BlockSpec(memory_space=pltpu.VMEM),
            pl.BlockSpec(memory_space=pltpu.VMEM),
            pl.BlockSpec(memory_space=pltpu.VMEM),
        ],
        out_specs=pl.BlockSpec(memory_space=pl.ANY),
        out_shape=jax.ShapeDtypeStruct((n, n, d), jnp.float32),
        scratch_shapes=[
            pltpu.VMEM((_REP, n, d), jnp.float32),
            pltpu.SemaphoreType.DMA,
        ],
    )(X, W, b.reshape(1, d))
    return out
